# baseline (speedup 1.0000x reference)
import jax
import jax.numpy as jnp
import numpy as np
from jax import lax
from jax.experimental import pallas as pl
from jax.experimental.pallas import tpu as pltpu
from jax.experimental.pallas import tpu_sc as plsc

D = 64
SCALE = 8.0
NC = 2
NS = 16
NW = NC * NS
SEQ = 50
BATCH = 16384
B_TOTAL = BATCH * SEQ
PER_W = B_TOTAL // NW
CHUNK = 256
NCHUNK = PER_W // CHUNK
LANES = 16
WBLK = BATCH // 128
SLABS = CHUNK // 128

VOCAB = 1000000
VT_FULL = VOCAB // 128
V_TAIL0 = VT_FULL * 128
WIDE = 80


def _widen_transpose(tall, wb, iota):
    iotaw = iota * WIDE
    @plsc.parallel_loop(0, 8 * LANES, unroll=4)
    def u_body(u):
        t = u // LANES
        k = u % LANES
        cvk = (iota + k) % 16
        vvv = iota + t * LANES
        wvv = iotaw + cvk + t * (LANES * WIDE)
        for c0 in range(0, D, LANES):
            v = plsc.load_gather(tall, [cvk + c0, vvv])
            plsc.store_scatter(wb, [wvv + c0], v * SCALE)


def _widen_body(lut_t_hbm, wide_hbm, tall0, tall1, wb0, wb1, l0, l1, s0, s1):
    wid = lax.axis_index("s") * NC + lax.axis_index("c")
    iota = lax.iota(jnp.int32, LANES)
    tall = (tall0, tall1)
    wb = (wb0, wb1)
    lsem = (l0, l1)
    ssem = (s0, s1)

    nb0 = VT_FULL // NW
    nw_extra = VT_FULL - nb0 * NW

    def vt_of(n):
        return wid + NW * n

    def start_load(n, b):
        pltpu.make_async_copy(
            lut_t_hbm.at[pl.ds(0, D), pl.ds(vt_of(n) * 128, 128)],
            tall[b], lsem[b]).start()

    def wait_load(b):
        pltpu.make_async_copy(
            lut_t_hbm.at[pl.ds(0, D), pl.ds(0, 128)], tall[b],
            lsem[b]).wait()

    def start_store(n, b):
        pltpu.make_async_copy(
            wb[b], wide_hbm.at[pl.ds(vt_of(n) * (128 * WIDE), 128 * WIDE)],
            ssem[b]).start()

    def wait_store(b):
        pltpu.make_async_copy(
            wb[b], wide_hbm.at[pl.ds(0, 128 * WIDE)], ssem[b]).wait()

    start_load(0, 0)

    def pair(j, _):
        for b in range(2):
            n = 2 * j + b
            wait_load(b)
            if b == 0:
                start_load(n + 1, 1)
            else:
                @pl.when((n + 1 < nb0) | (wid < nw_extra))
                def _():
                    start_load(n + 1, 0)

            @pl.when(n >= 2)
            def _():
                wait_store(b)
            _widen_transpose(tall[b], wb[b], iota)
            start_store(n, b)
        return 0
    lax.fori_loop(0, nb0 // 2, pair, 0)

    @pl.when(wid < nw_extra)
    def _():
        wait_load(0)
        wait_store(0)
        _widen_transpose(tall[0], wb[0], iota)
        start_store(nb0, 0)

    wait_store(1)
    wait_store(0)


def _widen(lut_t):
    mesh = plsc.VectorSubcoreMesh(core_axis_name="c", subcore_axis_name="s")
    return pl.kernel(
        _widen_body,
        out_type=jax.ShapeDtypeStruct((VOCAB * WIDE,), jnp.float32),
        mesh=mesh,
        compiler_params=pltpu.CompilerParams(
            use_tc_tiling_on_sc=True, needs_layout_passes=False),
        scratch_types=[
            pltpu.VMEM((D, 128), jnp.float32),
            pltpu.VMEM((D, 128), jnp.float32),
            pltpu.VMEM((128 * WIDE,), jnp.float32),
            pltpu.VMEM((128 * WIDE,), jnp.float32),
            pltpu.SemaphoreType.DMA,
            pltpu.SemaphoreType.DMA,
            pltpu.SemaphoreType.DMA,
            pltpu.SemaphoreType.DMA,
        ],
    )(lut_t)


def _transpose_scale(rows_b, out_t, iota):
    @plsc.parallel_loop(0, (CHUNK // LANES) * LANES, unroll=4)
    def u_body(u):
        t = u // LANES
        k = u % LANES
        s = t // 8
        rv = iota + t * LANES
        d2v = iota + (t % 8) * LANES
        cvk = (iota + k) % 16
        d0k = (cvk // 8) * SLABS
        d1v = cvk % 8
        for c0 in range(0, D, LANES):
            cv = cvk + c0
            v = plsc.load_gather(rows_b, [rv, cv])
            d0 = d0k + ((c0 // 8) * SLABS + s)
            plsc.store_scatter(out_t, [d0, d1v, d2v], v)


def _body(x_hbm, lut_hbm, out_hbm, idx_all, rows0, rows1, ot0, ot1,
          g0, g1, o0, o1):
    wid = lax.axis_index("s") * NC + lax.axis_index("c")
    base = wid * PER_W

    iota = lax.iota(jnp.int32, LANES)

    pltpu.sync_copy(x_hbm.at[pl.ds(base, PER_W)], idx_all)

    rows = (rows0, rows1)
    out_t = (ot0, ot1)
    sems = (g0, g1)
    osem = (o0, o1)

    def start_gather(ci, b):
        pltpu.make_async_copy(
            lut_hbm.at[idx_all.at[pl.ds(ci * CHUNK, CHUNK)]],
            rows[b], sems[b]).start()

    def wait_gather(b):
        pltpu.make_async_copy(
            lut_hbm.at[idx_all.at[pl.ds(0, CHUNK)]],
            rows[b], sems[b]).wait()

    def start_stores(ci, b):
        b0 = base + ci * CHUNK
        j = b0 // BATCH
        w0 = (b0 % BATCH) // 128
        for cb in range(D // 8):
            pltpu.make_async_copy(
                out_t[b].at[pl.ds(cb * SLABS, SLABS)],
                out_hbm.at[j, cb, pl.ds(w0, SLABS)], osem[b]).start()

    def wait_stores(b):
        for cb in range(D // 8):
            pltpu.make_async_copy(
                out_t[b].at[pl.ds(cb * SLABS, SLABS)],
                out_hbm.at[0, cb, pl.ds(0, SLABS)], osem[b]).wait()

    def process(ci, b, guard_stores):
        wait_gather(b)
        if guard_stores:
            @pl.when(ci >= 2)
            def _():
                wait_stores(b)
        _transpose_scale(rows[b], out_t[b], iota)
        start_stores(ci, b)

    start_gather(0, 0)
    start_gather(1, 1)

    def pair_body(jj, _):
        for b in range(2):
            ci = 2 * jj + b
            process(ci, b, True)
            start_gather(ci + 2, b)
        return 0
    lax.fori_loop(0, NCHUNK // 2 - 1, pair_body, 0)

    for b, ci in ((0, NCHUNK - 2), (1, NCHUNK - 1)):
        process(ci, b, True)
    wait_stores(0)
    wait_stores(1)


def _embed(x2_flat, lut2):
    mesh = plsc.VectorSubcoreMesh(core_axis_name="c", subcore_axis_name="s")
    return pl.kernel(
        _body,
        out_type=jax.ShapeDtypeStruct((SEQ, D // 8, WBLK, 8, 128),
                                      jnp.float32),
        mesh=mesh,
        compiler_params=pltpu.CompilerParams(
            use_tc_tiling_on_sc=False, needs_layout_passes=False),
        scratch_types=[
            pltpu.VMEM((PER_W,), jnp.int32),
            pltpu.VMEM((CHUNK, WIDE), jnp.float32),
            pltpu.VMEM((CHUNK, WIDE), jnp.float32),
            pltpu.VMEM(((D // 8) * SLABS, 8, 128), jnp.float32),
            pltpu.VMEM(((D // 8) * SLABS, 8, 128), jnp.float32),
            pltpu.SemaphoreType.DMA,
            pltpu.SemaphoreType.DMA,
            pltpu.SemaphoreType.DMA,
            pltpu.SemaphoreType.DMA,
        ],
    )(x2_flat, lut2)


def kernel(x, lut):
    wide = _widen(lut.T)
    tail = jnp.pad(lut[V_TAIL0:] * SCALE, ((0, 0), (0, WIDE - D)))
    wide = lax.dynamic_update_slice(wide, tail.reshape(-1),
                                    (V_TAIL0 * WIDE,))
    lut2 = wide.reshape(VOCAB, WIDE)
    x2 = x.T.reshape(-1).astype(jnp.int32)
    out5 = _embed(x2, lut2)
    return out5.transpose(2, 4, 0, 1, 3).reshape(BATCH, SEQ, D)


# speedup vs baseline: 1.0189x; 1.0189x over previous
import jax
import jax.numpy as jnp
import numpy as np
from jax import lax
from jax.experimental import pallas as pl
from jax.experimental.pallas import tpu as pltpu
from jax.experimental.pallas import tpu_sc as plsc

D = 64
SCALE = 8.0
NC = 2
NS = 16
NW = NC * NS
SEQ = 50
BATCH = 16384
B_TOTAL = BATCH * SEQ
PER_W = B_TOTAL // NW
CHUNK = 256
NCHUNK = PER_W // CHUNK
LANES = 16
WBLK = BATCH // 128
SLABS = CHUNK // 128

VOCAB = 1000000
VT_FULL = VOCAB // 128
V_TAIL0 = VT_FULL * 128


def _widen_transpose(tall, wb, iota):
    iota128 = iota * 128
    @plsc.parallel_loop(0, 8 * LANES, unroll=4)
    def u_body(u):
        t = u // LANES
        k = u % LANES
        cvk = (iota + k) % 16
        vvv = iota + t * LANES
        wvv = iota128 + cvk + t * 2048
        for c0 in range(0, D, LANES):
            v = plsc.load_gather(tall, [cvk + c0, vvv])
            plsc.store_scatter(wb, [wvv + c0], v * SCALE)


def _widen_body(lut_t_hbm, wide_hbm, tall0, tall1, wb0, wb1, l0, l1, s0, s1):
    wid = lax.axis_index("s") * NC + lax.axis_index("c")
    iota = lax.iota(jnp.int32, LANES)
    tall = (tall0, tall1)
    wb = (wb0, wb1)
    lsem = (l0, l1)
    ssem = (s0, s1)

    nb0 = VT_FULL // NW
    nw_extra = VT_FULL - nb0 * NW

    def vt_of(n):
        return wid + NW * n

    def start_load(n, b):
        pltpu.make_async_copy(
            lut_t_hbm.at[pl.ds(0, D), pl.ds(vt_of(n) * 128, 128)],
            tall[b], lsem[b]).start()

    def wait_load(b):
        pltpu.make_async_copy(
            lut_t_hbm.at[pl.ds(0, D), pl.ds(0, 128)], tall[b],
            lsem[b]).wait()

    def start_store(n, b):
        pltpu.make_async_copy(
            wb[b], wide_hbm.at[pl.ds(vt_of(n) * 16384, 16384)],
            ssem[b]).start()

    def wait_store(b):
        pltpu.make_async_copy(
            wb[b], wide_hbm.at[pl.ds(0, 16384)], ssem[b]).wait()

    start_load(0, 0)

    def pair(j, _):
        for b in range(2):
            n = 2 * j + b
            wait_load(b)
            if b == 0:
                start_load(n + 1, 1)
            else:
                @pl.when((n + 1 < nb0) | (wid < nw_extra))
                def _():
                    start_load(n + 1, 0)

            @pl.when(n >= 2)
            def _():
                wait_store(b)
            _widen_transpose(tall[b], wb[b], iota)
            start_store(n, b)
        return 0
    lax.fori_loop(0, nb0 // 2, pair, 0)

    @pl.when(wid < nw_extra)
    def _():
        wait_load(0)
        wait_store(0)
        _widen_transpose(tall[0], wb[0], iota)
        start_store(nb0, 0)

    wait_store(1)
    wait_store(0)


def _widen(lut_t):
    mesh = plsc.VectorSubcoreMesh(core_axis_name="c", subcore_axis_name="s")
    return pl.kernel(
        _widen_body,
        out_type=jax.ShapeDtypeStruct((VOCAB * 128,), jnp.float32),
        mesh=mesh,
        compiler_params=pltpu.CompilerParams(
            use_tc_tiling_on_sc=True, needs_layout_passes=False),
        scratch_types=[
            pltpu.VMEM((D, 128), jnp.float32),
            pltpu.VMEM((D, 128), jnp.float32),
            pltpu.VMEM((128 * 128,), jnp.float32),
            pltpu.VMEM((128 * 128,), jnp.float32),
            pltpu.SemaphoreType.DMA,
            pltpu.SemaphoreType.DMA,
            pltpu.SemaphoreType.DMA,
            pltpu.SemaphoreType.DMA,
        ],
    )(lut_t)


def _transpose_scale(rows_b, out_t, iota):
    @plsc.parallel_loop(0, (CHUNK // LANES) * LANES, unroll=4)
    def u_body(u):
        t = u // LANES
        k = u % LANES
        s = t // 8
        rv = iota + t * LANES
        d2v = iota + (t % 8) * LANES
        cvk = (iota + k) % 16
        d0k = (cvk // 8) * SLABS
        d1v = cvk % 8
        for c0 in range(0, D, LANES):
            cv = cvk + c0
            v = plsc.load_gather(rows_b, [rv, cv])
            d0 = d0k + ((c0 // 8) * SLABS + s)
            plsc.store_scatter(out_t, [d0, d1v, d2v], v)


def _body(x_hbm, lut_hbm, out_hbm, idx_all, rows0, rows1, ot0, ot1,
          g0, g1, o0, o1):
    wid = lax.axis_index("s") * NC + lax.axis_index("c")
    base = wid * PER_W

    iota = lax.iota(jnp.int32, LANES)

    pltpu.sync_copy(x_hbm.at[pl.ds(base, PER_W)], idx_all)

    rows = (rows0, rows1)
    out_t = (ot0, ot1)
    sems = (g0, g1)
    osem = (o0, o1)

    def start_gather(ci, b):
        pltpu.make_async_copy(
            lut_hbm.at[idx_all.at[pl.ds(ci * CHUNK, CHUNK)]],
            rows[b], sems[b]).start()

    def wait_gather(b):
        pltpu.make_async_copy(
            lut_hbm.at[idx_all.at[pl.ds(0, CHUNK)]],
            rows[b], sems[b]).wait()

    def start_stores(ci, b):
        b0 = base + ci * CHUNK
        j = b0 // BATCH
        w0 = (b0 % BATCH) // 128
        for cb in range(D // 8):
            pltpu.make_async_copy(
                out_t[b].at[pl.ds(cb * SLABS, SLABS)],
                out_hbm.at[j, cb, pl.ds(w0, SLABS)], osem[b]).start()

    def wait_stores(b):
        for cb in range(D // 8):
            pltpu.make_async_copy(
                out_t[b].at[pl.ds(cb * SLABS, SLABS)],
                out_hbm.at[0, cb, pl.ds(0, SLABS)], osem[b]).wait()

    def process(ci, b, guard_stores):
        wait_gather(b)
        if guard_stores:
            @pl.when(ci >= 2)
            def _():
                wait_stores(b)
        _transpose_scale(rows[b], out_t[b], iota)
        start_stores(ci, b)

    start_gather(0, 0)
    start_gather(1, 1)

    def pair_body(jj, _):
        for b in range(2):
            ci = 2 * jj + b
            process(ci, b, True)
            start_gather(ci + 2, b)
        return 0
    lax.fori_loop(0, NCHUNK // 2 - 1, pair_body, 0)

    for b, ci in ((0, NCHUNK - 2), (1, NCHUNK - 1)):
        process(ci, b, True)
    wait_stores(0)
    wait_stores(1)


def _embed(x2_flat, lut2):
    mesh = plsc.VectorSubcoreMesh(core_axis_name="c", subcore_axis_name="s")
    return pl.kernel(
        _body,
        out_type=jax.ShapeDtypeStruct((SEQ, D // 8, WBLK, 8, 128),
                                      jnp.float32),
        mesh=mesh,
        compiler_params=pltpu.CompilerParams(
            use_tc_tiling_on_sc=False, needs_layout_passes=False),
        scratch_types=[
            pltpu.VMEM((PER_W,), jnp.int32),
            pltpu.VMEM((CHUNK, D), jnp.float32),
            pltpu.VMEM((CHUNK, D), jnp.float32),
            pltpu.VMEM(((D // 8) * SLABS, 8, 128), jnp.float32),
            pltpu.VMEM(((D // 8) * SLABS, 8, 128), jnp.float32),
            pltpu.SemaphoreType.DMA,
            pltpu.SemaphoreType.DMA,
            pltpu.SemaphoreType.DMA,
            pltpu.SemaphoreType.DMA,
        ],
    )(x2_flat, lut2)


def kernel(x, lut):
    wide = _widen(lut.T)
    tail = jnp.pad(lut[V_TAIL0:] * SCALE, ((0, 0), (0, 128 - D)))
    wide = lax.dynamic_update_slice(wide, tail.reshape(-1), (V_TAIL0 * 128,))
    lut2 = wide.reshape(2 * VOCAB, D)
    x2 = (x.T.reshape(-1) * 2).astype(jnp.int32)
    out5 = _embed(x2, lut2)
    return out5.transpose(2, 4, 0, 1, 3).reshape(BATCH, SEQ, D)
